# baseline (speedup 1.0000x reference)
import functools

import jax
import jax.numpy as jnp
from jax import lax
from jax.experimental import pallas as pl
from jax.experimental.pallas import tpu as pltpu
from jax.experimental.pallas import tpu_sc as plsc

N = 10000
E = 160000
D = 256
H = 512
OUT = 256

NC = 2
NS = 16
LANE = 128

EDGES_PER_TILE = E // NS
K = 80
NCH = EDGES_PER_TILE // K
NGRP = 5
M = NCH // NGRP
NB = 3
NPAD = 10240
ROWS_PER_TILE = NPAD // NS
ZR = 16


def _sc_segment_sum(cb):
    cbh = cb // NC

    mesh = plsc.VectorSubcoreMesh(core_axis_name="c", subcore_axis_name="s",
                                  num_cores=NC, num_subcores=NS)

    @functools.partial(
        pl.kernel,
        out_type=jax.ShapeDtypeStruct((cb * NPAD, LANE), jnp.float32),
        mesh=mesh,
        scratch_types=[
            pltpu.VMEM((M, K), jnp.int32),
            pltpu.VMEM((M, K), jnp.int32),
            pltpu.VMEM((NB, K, LANE), jnp.float32),
            pltpu.VMEM((ZR, LANE), jnp.float32),
            pltpu.VMEM_SHARED((NPAD, LANE), jnp.float32),
            [pltpu.SemaphoreType.DMA] * NB,
            [pltpu.SemaphoreType.DMA] * NB,
        ],
    )
    def seg_kernel(ftab, sidx, didx, agg_out, src_v, dst_v, rows_v, zero_v,
                   agg_s, gs, ss):
        c = lax.axis_index("c")
        s = lax.axis_index("s")

        def zrow(i, _):
            def zcol(k, _):
                zero_v[i, pl.ds(k * 16, 16)] = jnp.zeros((16,), jnp.float32)
                return 0
            return lax.fori_loop(0, LANE // 16, zcol, 0)
        lax.fori_loop(0, ZR, zrow, 0)

        for b_i in range(cbh):
            idx_row = (c * cbh + b_i) * NS + s

            for z in range(ROWS_PER_TILE // ZR):
                pltpu.sync_copy(
                    zero_v, agg_s.at[pl.ds(s * ROWS_PER_TILE + z * ZR, ZR)])
            plsc.subcore_barrier()

            for g in range(NGRP):
                pltpu.sync_copy(sidx.at[idx_row * NGRP + g], src_v)
                pltpu.sync_copy(didx.at[s * NGRP + g], dst_v)

                for b in range(NB):
                    pltpu.async_copy(
                        ftab.at[src_v.at[b]], rows_v.at[b], gs[b])

                def rnd(i, _):
                    jb = NB * i
                    for b in range(NB):
                        pltpu.make_async_copy(
                            ftab.at[src_v.at[0]], rows_v.at[b], gs[b]).wait()
                        pltpu.async_copy(
                            rows_v.at[b], agg_s.at[dst_v.at[jb + b]], ss[b],
                            add=True)
                    for b in range(NB):
                        pltpu.make_async_copy(
                            rows_v.at[b], agg_s.at[dst_v.at[0]], ss[b]).wait()
                        jn = jnp.minimum(jb + NB + b, M - 1)
                        pltpu.async_copy(
                            ftab.at[src_v.at[jn]], rows_v.at[b], gs[b])
                    return 0
                lax.fori_loop(0, (M - 1) // NB, rnd, 0)

                pltpu.make_async_copy(
                    ftab.at[src_v.at[0]], rows_v.at[0], gs[0]).wait()
                pltpu.sync_copy(rows_v.at[0], agg_s.at[dst_v.at[M - 1]],
                                add=True)
                for b in range(1, NB):
                    pltpu.make_async_copy(
                        ftab.at[src_v.at[0]], rows_v.at[b], gs[b]).wait()
            plsc.subcore_barrier()

            blk = c + NC * b_i
            pltpu.sync_copy(
                agg_s.at[pl.ds(s * ROWS_PER_TILE, ROWS_PER_TILE)],
                agg_out.at[pl.ds(blk * NPAD + s * ROWS_PER_TILE,
                                 ROWS_PER_TILE)])
            if b_i + 1 < cbh:
                plsc.subcore_barrier()

    return seg_kernel


def _tc_mlp(cb_in, d_in, h_out, cb_out, rows):
    grid = (N // rows,)

    def body(agg_ref, h_ref, w1_ref, b1_ref, w2_ref, b2_ref, out_ref):
        acc = jnp.broadcast_to(b1_ref[0], (rows, H))
        for b in range(cb_in):
            acc = acc + jnp.dot(agg_ref[b],
                                w1_ref[b * LANE:(b + 1) * LANE, :],
                                preferred_element_type=jnp.float32)
        for b in range(cb_in):
            acc = acc + jnp.dot(h_ref[b],
                                w1_ref[d_in + b * LANE:d_in + (b + 1) * LANE, :],
                                preferred_element_type=jnp.float32)
        y = jnp.maximum(acc, 0.0)
        out = jnp.dot(y, w2_ref[...],
                      preferred_element_type=jnp.float32) + b2_ref[0]
        if cb_out:
            for b in range(cb_out):
                out_ref[b] = out[:, b * LANE:(b + 1) * LANE]
        else:
            out_ref[...] = out

    if cb_out:
        out_shape = jax.ShapeDtypeStruct((cb_out, N, LANE), jnp.float32)
        out_spec = pl.BlockSpec((cb_out, rows, LANE), lambda i: (0, i, 0))
    else:
        out_shape = jax.ShapeDtypeStruct((N, h_out), jnp.float32)
        out_spec = pl.BlockSpec((rows, h_out), lambda i: (i, 0))

    return pl.pallas_call(
        body,
        grid=grid,
        in_specs=[
            pl.BlockSpec((cb_in, rows, LANE), lambda i: (0, i, 0)),
            pl.BlockSpec((cb_in, rows, LANE), lambda i: (0, i, 0)),
            pl.BlockSpec((2 * d_in, H), lambda i: (0, 0)),
            pl.BlockSpec((1, H), lambda i: (0, 0)),
            pl.BlockSpec((H, h_out), lambda i: (0, 0)),
            pl.BlockSpec((1, h_out), lambda i: (0, 0)),
        ],
        out_specs=out_spec,
        out_shape=out_shape,
    )


def kernel(x, edge_index, W1_0, b1_0, W2_0, b2_0, W1_1, b1_1, W2_1, b2_1,
           W1_2, b1_2, W2_2, b2_2):
    src = edge_index[0].reshape(NS, NCH, K)
    dst = edge_index[1].reshape(NS, NCH, K)

    sidx2 = jnp.concatenate([src, src + N], axis=0).reshape(-1, M, K)
    sidx4 = jnp.concatenate([src, src + 2 * N, src + N, src + 3 * N],
                            axis=0).reshape(-1, M, K)
    dst = dst.reshape(-1, M, K)

    x_t = x.reshape(N, 2, LANE).transpose(1, 0, 2).reshape(2 * N, LANE)

    seg2 = _sc_segment_sum(2)
    seg4 = _sc_segment_sum(4)
    mlp0 = _tc_mlp(2, D, H, 4, 1000)
    mlp1 = _tc_mlp(4, H, H, 4, 1000)
    mlp2 = _tc_mlp(4, H, OUT, 0, 1000)

    agg0 = seg2(x_t, sidx2, dst)
    h1 = mlp0(agg0.reshape(2, NPAD, LANE), x_t.reshape(2, N, LANE),
              W1_0, b1_0.reshape(1, H), W2_0, b2_0.reshape(1, H))
    agg1 = seg4(h1.reshape(4 * N, LANE), sidx4, dst)
    h2 = mlp1(agg1.reshape(4, NPAD, LANE), h1,
              W1_1, b1_1.reshape(1, H), W2_1, b2_1.reshape(1, H))
    agg2 = seg4(h2.reshape(4 * N, LANE), sidx4, dst)
    out = mlp2(agg2.reshape(4, NPAD, LANE), h2,
               W1_2, b1_2.reshape(1, H), W2_2, b2_2.reshape(1, OUT))
    return out


# speedup vs baseline: 1.0350x; 1.0350x over previous
import functools

import jax
import jax.numpy as jnp
from jax import lax
from jax.experimental import pallas as pl
from jax.experimental.pallas import tpu as pltpu
from jax.experimental.pallas import tpu_sc as plsc

N = 10000
E = 160000
D = 256
H = 512
OUT = 256

NC = 2
NS = 16
LANE = 128

EDGES_PER_TILE = E // NS
K = 125
NCH = EDGES_PER_TILE // K
NGRP = 2
M = NCH // NGRP
NPAD = 10240
ROWS_PER_TILE = NPAD // NS
ZR = 16


def _sc_segment_sum(cb):
    cbh = cb // NC

    mesh = plsc.VectorSubcoreMesh(core_axis_name="c", subcore_axis_name="s",
                                  num_cores=NC, num_subcores=NS)

    @functools.partial(
        pl.kernel,
        out_type=jax.ShapeDtypeStruct((cb * NPAD, LANE), jnp.float32),
        mesh=mesh,
        scratch_types=[
            pltpu.VMEM((M, K), jnp.int32),
            pltpu.VMEM((M, K), jnp.int32),
            pltpu.VMEM((2, K, LANE), jnp.float32),
            pltpu.VMEM((ZR, LANE), jnp.float32),
            pltpu.VMEM_SHARED((NPAD, LANE), jnp.float32),
            pltpu.SemaphoreType.DMA,
            pltpu.SemaphoreType.DMA,
        ],
    )
    def seg_kernel(ftab, sidx, didx, agg_out, src_v, dst_v, rows_v, zero_v,
                   agg_s, gsA, gsB):
        c = lax.axis_index("c")
        s = lax.axis_index("s")

        def zrow(i, _):
            def zcol(k, _):
                zero_v[i, pl.ds(k * 16, 16)] = jnp.zeros((16,), jnp.float32)
                return 0
            return lax.fori_loop(0, LANE // 16, zcol, 0)
        lax.fori_loop(0, ZR, zrow, 0)

        for b_i in range(cbh):
            idx_row = (c * cbh + b_i) * NS + s

            for z in range(ROWS_PER_TILE // ZR):
                pltpu.sync_copy(
                    zero_v, agg_s.at[pl.ds(s * ROWS_PER_TILE + z * ZR, ZR)])
            plsc.subcore_barrier()

            for g in range(NGRP):
                pltpu.sync_copy(sidx.at[idx_row * NGRP + g], src_v)
                pltpu.sync_copy(didx.at[s * NGRP + g], dst_v)

                bufA = rows_v.at[0]
                bufB = rows_v.at[1]
                pltpu.async_copy(ftab.at[src_v.at[0]], bufA, gsA)

                def pair(i, _):
                    j = 2 * i
                    pltpu.make_async_copy(
                        ftab.at[src_v.at[0]], bufA, gsA).wait()
                    pltpu.async_copy(ftab.at[src_v.at[j + 1]], bufB, gsB)
                    pltpu.sync_copy(bufA, agg_s.at[dst_v.at[j]], add=True)
                    pltpu.make_async_copy(
                        ftab.at[src_v.at[0]], bufB, gsB).wait()
                    jn = jnp.minimum(j + 2, M - 1)
                    pltpu.async_copy(ftab.at[src_v.at[jn]], bufA, gsA)
                    pltpu.sync_copy(bufB, agg_s.at[dst_v.at[j + 1]], add=True)
                    return 0
                lax.fori_loop(0, M // 2, pair, 0)
                pltpu.make_async_copy(ftab.at[src_v.at[0]], bufA, gsA).wait()
            plsc.subcore_barrier()

            blk = c + NC * b_i
            pltpu.sync_copy(
                agg_s.at[pl.ds(s * ROWS_PER_TILE, ROWS_PER_TILE)],
                agg_out.at[pl.ds(blk * NPAD + s * ROWS_PER_TILE,
                                 ROWS_PER_TILE)])
            if b_i + 1 < cbh:
                plsc.subcore_barrier()

    return seg_kernel


def _tc_self(cb_in, d_in, rows):
    def body(h_ref, w1b_ref, b1_ref, out_ref):
        acc = jnp.broadcast_to(b1_ref[0], (rows, H))
        for b in range(cb_in):
            acc = acc + jnp.dot(h_ref[b],
                                w1b_ref[b * LANE:(b + 1) * LANE, :],
                                preferred_element_type=jnp.float32)
        out_ref[...] = acc

    return pl.pallas_call(
        body,
        grid=(N // rows,),
        in_specs=[
            pl.BlockSpec((cb_in, rows, LANE), lambda i: (0, i, 0)),
            pl.BlockSpec((d_in, H), lambda i: (0, 0)),
            pl.BlockSpec((1, H), lambda i: (0, 0)),
        ],
        out_specs=pl.BlockSpec((rows, H), lambda i: (i, 0)),
        out_shape=jax.ShapeDtypeStruct((N, H), jnp.float32),
    )


def _tc_final(cb_in, d_in, h_out, cb_out, rows):
    def body(agg_ref, self_ref, w1a_ref, w2_ref, b2_ref, out_ref):
        acc = self_ref[...]
        for b in range(cb_in):
            acc = acc + jnp.dot(agg_ref[b],
                                w1a_ref[b * LANE:(b + 1) * LANE, :],
                                preferred_element_type=jnp.float32)
        y = jnp.maximum(acc, 0.0)
        out = jnp.dot(y, w2_ref[...],
                      preferred_element_type=jnp.float32) + b2_ref[0]
        if cb_out:
            for b in range(cb_out):
                out_ref[b] = out[:, b * LANE:(b + 1) * LANE]
        else:
            out_ref[...] = out

    if cb_out:
        out_shape = jax.ShapeDtypeStruct((cb_out, N, LANE), jnp.float32)
        out_spec = pl.BlockSpec((cb_out, rows, LANE), lambda i: (0, i, 0))
    else:
        out_shape = jax.ShapeDtypeStruct((N, h_out), jnp.float32)
        out_spec = pl.BlockSpec((rows, h_out), lambda i: (i, 0))

    return pl.pallas_call(
        body,
        grid=(N // rows,),
        in_specs=[
            pl.BlockSpec((cb_in, rows, LANE), lambda i: (0, i, 0)),
            pl.BlockSpec((rows, H), lambda i: (i, 0)),
            pl.BlockSpec((d_in, H), lambda i: (0, 0)),
            pl.BlockSpec((H, h_out), lambda i: (0, 0)),
            pl.BlockSpec((1, h_out), lambda i: (0, 0)),
        ],
        out_specs=out_spec,
        out_shape=out_shape,
    )


def kernel(x, edge_index, W1_0, b1_0, W2_0, b2_0, W1_1, b1_1, W2_1, b2_1,
           W1_2, b1_2, W2_2, b2_2):
    src = edge_index[0].reshape(NS, NCH, K)
    dst = edge_index[1].reshape(NS, NCH, K)

    sidx2 = jnp.concatenate([src, src + N], axis=0).reshape(-1, M, K)
    sidx4 = jnp.concatenate([src, src + 2 * N, src + N, src + 3 * N],
                            axis=0).reshape(-1, M, K)
    dst = dst.reshape(-1, M, K)

    x_t = x.reshape(N, 2, LANE).transpose(1, 0, 2).reshape(2 * N, LANE)

    seg2 = _sc_segment_sum(2)
    seg4 = _sc_segment_sum(4)
    self0 = _tc_self(2, D, 1000)
    self1 = _tc_self(4, H, 1000)
    fin0 = _tc_final(2, D, H, 4, 1000)
    fin1 = _tc_final(4, H, H, 4, 1000)
    fin2 = _tc_final(4, H, OUT, 0, 1000)

    xb = x_t.reshape(2, N, LANE)
    agg0 = seg2(x_t, sidx2, dst)
    acc0 = self0(xb, W1_0[D:], b1_0.reshape(1, H))
    h1 = fin0(agg0.reshape(2, NPAD, LANE), acc0, W1_0[:D], W2_0,
              b2_0.reshape(1, H))
    agg1 = seg4(h1.reshape(4 * N, LANE), sidx4, dst)
    acc1 = self1(h1, W1_1[H:], b1_1.reshape(1, H))
    h2 = fin1(agg1.reshape(4, NPAD, LANE), acc1, W1_1[:H], W2_1,
              b2_1.reshape(1, H))
    agg2 = seg4(h2.reshape(4 * N, LANE), sidx4, dst)
    acc2 = self1(h2, W1_2[H:], b1_2.reshape(1, H))
    out = fin2(agg2.reshape(4, NPAD, LANE), acc2, W1_2[:H], W2_2,
               b2_2.reshape(1, OUT))
    return out


# speedup vs baseline: 1.0484x; 1.0129x over previous
import functools

import jax
import jax.numpy as jnp
from jax import lax
from jax.experimental import pallas as pl
from jax.experimental.pallas import tpu as pltpu
from jax.experimental.pallas import tpu_sc as plsc

N = 10000
E = 160000
D = 256
H = 512
OUT = 256

NC = 2
NS = 16
LANE = 128

EDGES_PER_TILE = E // NS
K = 125
NCH = EDGES_PER_TILE // K
NGRP = 2
M = NCH // NGRP
NPAD = 10240
ROWS_PER_TILE = NPAD // NS
ZR = 32


def _sc_segment_sum(cb):
    cbh = cb // NC

    mesh = plsc.VectorSubcoreMesh(core_axis_name="c", subcore_axis_name="s",
                                  num_cores=NC, num_subcores=NS)

    @functools.partial(
        pl.kernel,
        out_type=jax.ShapeDtypeStruct((cb * NPAD, LANE), jnp.float32),
        mesh=mesh,
        scratch_types=[
            pltpu.VMEM((M, K), jnp.int32),
            pltpu.VMEM((M, K), jnp.int32),
            pltpu.VMEM((2, K, LANE), jnp.float32),
            pltpu.VMEM((ZR, LANE), jnp.float32),
            pltpu.VMEM_SHARED((NPAD, LANE), jnp.float32),
            pltpu.SemaphoreType.DMA,
            pltpu.SemaphoreType.DMA,
        ],
    )
    def seg_kernel(ftab, sidx, didx, agg_out, src_v, dst_v, rows_v, zero_v,
                   agg_s, gsA, gsB):
        c = lax.axis_index("c")
        s = lax.axis_index("s")

        def zrow(i, _):
            def zcol(k, _):
                zero_v[i, pl.ds(k * 16, 16)] = jnp.zeros((16,), jnp.float32)
                return 0
            return lax.fori_loop(0, LANE // 16, zcol, 0)
        lax.fori_loop(0, ZR, zrow, 0)

        for b_i in range(cbh):
            idx_row = (c * cbh + b_i) * NS + s

            for z in range(ROWS_PER_TILE // ZR):
                pltpu.async_copy(
                    zero_v, agg_s.at[pl.ds(s * ROWS_PER_TILE + z * ZR, ZR)],
                    gsA)
            for z in range(ROWS_PER_TILE // ZR):
                pltpu.make_async_copy(
                    zero_v, agg_s.at[pl.ds(s * ROWS_PER_TILE, ZR)],
                    gsA).wait()
            plsc.subcore_barrier()

            for g in range(NGRP):
                pltpu.sync_copy(sidx.at[idx_row * NGRP + g], src_v)
                pltpu.sync_copy(didx.at[s * NGRP + g], dst_v)

                bufA = rows_v.at[0]
                bufB = rows_v.at[1]
                pltpu.async_copy(ftab.at[src_v.at[0]], bufA, gsA)

                def pair(i, _):
                    j = 2 * i
                    pltpu.make_async_copy(
                        ftab.at[src_v.at[0]], bufA, gsA).wait()
                    pltpu.async_copy(ftab.at[src_v.at[j + 1]], bufB, gsB)
                    pltpu.sync_copy(bufA, agg_s.at[dst_v.at[j]], add=True)
                    pltpu.make_async_copy(
                        ftab.at[src_v.at[0]], bufB, gsB).wait()
                    jn = jnp.minimum(j + 2, M - 1)
                    pltpu.async_copy(ftab.at[src_v.at[jn]], bufA, gsA)
                    pltpu.sync_copy(bufB, agg_s.at[dst_v.at[j + 1]], add=True)
                    return 0
                lax.fori_loop(0, M // 2, pair, 0)
                pltpu.make_async_copy(ftab.at[src_v.at[0]], bufA, gsA).wait()
            plsc.subcore_barrier()

            blk = c + NC * b_i
            pltpu.sync_copy(
                agg_s.at[pl.ds(s * ROWS_PER_TILE, ROWS_PER_TILE)],
                agg_out.at[pl.ds(blk * NPAD + s * ROWS_PER_TILE,
                                 ROWS_PER_TILE)])
            if b_i + 1 < cbh:
                plsc.subcore_barrier()

    return seg_kernel


def _tc_mlp(cb_in, d_in, h_out, cb_out, rows):
    def body(agg_ref, h_ref, w1_ref, b1_ref, w2_ref, b2_ref, out_ref):
        acc = jnp.broadcast_to(b1_ref[0], (rows, H))
        for b in range(cb_in):
            acc = acc + jnp.dot(agg_ref[b],
                                w1_ref[b * LANE:(b + 1) * LANE, :],
                                preferred_element_type=jnp.float32)
        for b in range(cb_in):
            acc = acc + jnp.dot(h_ref[b],
                                w1_ref[d_in + b * LANE:d_in + (b + 1) * LANE, :],
                                preferred_element_type=jnp.float32)
        y = jnp.maximum(acc, 0.0)
        out = jnp.dot(y, w2_ref[...],
                      preferred_element_type=jnp.float32) + b2_ref[0]
        if cb_out:
            for b in range(cb_out):
                out_ref[b] = out[:, b * LANE:(b + 1) * LANE]
        else:
            out_ref[...] = out

    if cb_out:
        out_shape = jax.ShapeDtypeStruct((cb_out, N, LANE), jnp.float32)
        out_spec = pl.BlockSpec((cb_out, rows, LANE), lambda i: (0, i, 0))
    else:
        out_shape = jax.ShapeDtypeStruct((N, h_out), jnp.float32)
        out_spec = pl.BlockSpec((rows, h_out), lambda i: (i, 0))

    return pl.pallas_call(
        body,
        grid=(N // rows,),
        in_specs=[
            pl.BlockSpec((cb_in, rows, LANE), lambda i: (0, i, 0)),
            pl.BlockSpec((cb_in, rows, LANE), lambda i: (0, i, 0)),
            pl.BlockSpec((2 * d_in, H), lambda i: (0, 0)),
            pl.BlockSpec((1, H), lambda i: (0, 0)),
            pl.BlockSpec((H, h_out), lambda i: (0, 0)),
            pl.BlockSpec((1, h_out), lambda i: (0, 0)),
        ],
        out_specs=out_spec,
        out_shape=out_shape,
    )


def kernel(x, edge_index, W1_0, b1_0, W2_0, b2_0, W1_1, b1_1, W2_1, b2_1,
           W1_2, b1_2, W2_2, b2_2):
    src = edge_index[0].reshape(NS, NCH, K)
    dst = edge_index[1].reshape(NS, NCH, K)

    sidx2 = jnp.concatenate([src, src + N], axis=0).reshape(-1, M, K)
    sidx4 = jnp.concatenate([src, src + 2 * N, src + N, src + 3 * N],
                            axis=0).reshape(-1, M, K)
    dst = dst.reshape(-1, M, K)

    x_t = x.reshape(N, 2, LANE).transpose(1, 0, 2).reshape(2 * N, LANE)

    seg2 = _sc_segment_sum(2)
    seg4 = _sc_segment_sum(4)
    mlp0 = _tc_mlp(2, D, H, 4, 1000)
    mlp1 = _tc_mlp(4, H, H, 4, 1000)
    mlp2 = _tc_mlp(4, H, OUT, 0, 1000)

    agg0 = seg2(x_t, sidx2, dst)
    h1 = mlp0(agg0.reshape(2, NPAD, LANE), x_t.reshape(2, N, LANE),
              W1_0, b1_0.reshape(1, H), W2_0, b2_0.reshape(1, H))
    agg1 = seg4(h1.reshape(4 * N, LANE), sidx4, dst)
    h2 = mlp1(agg1.reshape(4, NPAD, LANE), h1,
              W1_1, b1_1.reshape(1, H), W2_1, b2_1.reshape(1, H))
    agg2 = seg4(h2.reshape(4 * N, LANE), sidx4, dst)
    out = mlp2(agg2.reshape(4, NPAD, LANE), h2,
               W1_2, b1_2.reshape(1, H), W2_2, b2_2.reshape(1, OUT))
    return out


# speedup vs baseline: 1.0485x; 1.0001x over previous
import functools

import jax
import jax.numpy as jnp
from jax import lax
from jax.experimental import pallas as pl
from jax.experimental.pallas import tpu as pltpu
from jax.experimental.pallas import tpu_sc as plsc

N = 10000
E = 160000
D = 256
H = 512
OUT = 256

NC = 2
NS = 16
LANE = 128

EDGES_PER_TILE = E // NS
K = 125
NCH = EDGES_PER_TILE // K
NGRP = 2
M = NCH // NGRP
NPAD = 10240
ROWS_PER_TILE = NPAD // NS
ZR = 32


def _sc_segment_sum(cb):
    cbh = cb // NC

    mesh = plsc.VectorSubcoreMesh(core_axis_name="c", subcore_axis_name="s",
                                  num_cores=NC, num_subcores=NS)

    @functools.partial(
        pl.kernel,
        out_type=jax.ShapeDtypeStruct((cb * NPAD, LANE), jnp.float32),
        mesh=mesh,
        scratch_types=[
            pltpu.VMEM((M, K), jnp.int32),
            pltpu.VMEM((M, K), jnp.int32),
            pltpu.VMEM((2, K, LANE), jnp.float32),
            pltpu.VMEM((ZR, LANE), jnp.float32),
            pltpu.VMEM_SHARED((NPAD, LANE), jnp.float32),
            pltpu.SemaphoreType.DMA,
            pltpu.SemaphoreType.DMA,
        ],
    )
    def seg_kernel(ftab, sidx, didx, agg_out, src_v, dst_v, rows_v, zero_v,
                   agg_s, gsA, gsB):
        c = lax.axis_index("c")
        s = lax.axis_index("s")

        def zrow(i, _):
            def zcol(k, _):
                zero_v[i, pl.ds(k * 16, 16)] = jnp.zeros((16,), jnp.float32)
                return 0
            return lax.fori_loop(0, LANE // 16, zcol, 0)
        lax.fori_loop(0, ZR, zrow, 0)

        for b_i in range(cbh):
            idx_row = (c * cbh + b_i) * NS + s

            for z in range(ROWS_PER_TILE // ZR):
                pltpu.async_copy(
                    zero_v, agg_s.at[pl.ds(s * ROWS_PER_TILE + z * ZR, ZR)],
                    gsA)
            for z in range(ROWS_PER_TILE // ZR):
                pltpu.make_async_copy(
                    zero_v, agg_s.at[pl.ds(s * ROWS_PER_TILE, ZR)],
                    gsA).wait()
            plsc.subcore_barrier()

            for g in range(NGRP):
                pltpu.sync_copy(sidx.at[idx_row * NGRP + g], src_v)
                pltpu.sync_copy(didx.at[s * NGRP + g], dst_v)

                bufA = rows_v.at[0]
                bufB = rows_v.at[1]
                pltpu.async_copy(ftab.at[src_v.at[0]], bufA, gsA)

                def pair(i, _):
                    j = 2 * i
                    pltpu.make_async_copy(
                        ftab.at[src_v.at[0]], bufA, gsA).wait()
                    pltpu.async_copy(ftab.at[src_v.at[j + 1]], bufB, gsB)
                    pltpu.sync_copy(bufA, agg_s.at[dst_v.at[j]], add=True)
                    pltpu.make_async_copy(
                        ftab.at[src_v.at[0]], bufB, gsB).wait()
                    jn = jnp.minimum(j + 2, M - 1)
                    pltpu.async_copy(ftab.at[src_v.at[jn]], bufA, gsA)
                    pltpu.sync_copy(bufB, agg_s.at[dst_v.at[j + 1]], add=True)
                    return 0
                lax.fori_loop(0, M // 2, pair, 0)
                pltpu.make_async_copy(ftab.at[src_v.at[0]], bufA, gsA).wait()
            plsc.subcore_barrier()

            blk = c + NC * b_i
            pltpu.sync_copy(
                agg_s.at[pl.ds(s * ROWS_PER_TILE, ROWS_PER_TILE)],
                agg_out.at[pl.ds(blk * NPAD + s * ROWS_PER_TILE,
                                 ROWS_PER_TILE)])
            if b_i + 1 < cbh:
                plsc.subcore_barrier()

    return seg_kernel


def _tc_mlp(cb_in, d_in, h_out, cb_out, rows):
    def body(agg_ref, h_ref, w1_ref, b1_ref, w2_ref, b2_ref, out_ref):
        bf = jnp.bfloat16
        acc = jnp.broadcast_to(b1_ref[0], (rows, H))
        for b in range(cb_in):
            acc = acc + jnp.dot(agg_ref[b].astype(bf),
                                w1_ref[b * LANE:(b + 1) * LANE, :].astype(bf),
                                preferred_element_type=jnp.float32)
        for b in range(cb_in):
            acc = acc + jnp.dot(
                h_ref[b].astype(bf),
                w1_ref[d_in + b * LANE:d_in + (b + 1) * LANE, :].astype(bf),
                preferred_element_type=jnp.float32)
        y = jnp.maximum(acc, 0.0)
        out = jnp.dot(y.astype(bf), w2_ref[...].astype(bf),
                      preferred_element_type=jnp.float32) + b2_ref[0]
        if cb_out:
            for b in range(cb_out):
                out_ref[b] = out[:, b * LANE:(b + 1) * LANE]
        else:
            out_ref[...] = out

    if cb_out:
        out_shape = jax.ShapeDtypeStruct((cb_out, N, LANE), jnp.float32)
        out_spec = pl.BlockSpec((cb_out, rows, LANE), lambda i: (0, i, 0))
    else:
        out_shape = jax.ShapeDtypeStruct((N, h_out), jnp.float32)
        out_spec = pl.BlockSpec((rows, h_out), lambda i: (i, 0))

    return pl.pallas_call(
        body,
        grid=(N // rows,),
        in_specs=[
            pl.BlockSpec((cb_in, rows, LANE), lambda i: (0, i, 0)),
            pl.BlockSpec((cb_in, rows, LANE), lambda i: (0, i, 0)),
            pl.BlockSpec((2 * d_in, H), lambda i: (0, 0)),
            pl.BlockSpec((1, H), lambda i: (0, 0)),
            pl.BlockSpec((H, h_out), lambda i: (0, 0)),
            pl.BlockSpec((1, h_out), lambda i: (0, 0)),
        ],
        out_specs=out_spec,
        out_shape=out_shape,
    )


def kernel(x, edge_index, W1_0, b1_0, W2_0, b2_0, W1_1, b1_1, W2_1, b2_1,
           W1_2, b1_2, W2_2, b2_2):
    src = edge_index[0].reshape(NS, NCH, K)
    dst = edge_index[1].reshape(NS, NCH, K)

    sidx2 = jnp.concatenate([src, src + N], axis=0).reshape(-1, M, K)
    sidx4 = jnp.concatenate([src, src + 2 * N, src + N, src + 3 * N],
                            axis=0).reshape(-1, M, K)
    dst = dst.reshape(-1, M, K)

    x_t = x.reshape(N, 2, LANE).transpose(1, 0, 2).reshape(2 * N, LANE)

    seg2 = _sc_segment_sum(2)
    seg4 = _sc_segment_sum(4)
    mlp0 = _tc_mlp(2, D, H, 4, 1000)
    mlp1 = _tc_mlp(4, H, H, 4, 1000)
    mlp2 = _tc_mlp(4, H, OUT, 0, 1000)

    agg0 = seg2(x_t, sidx2, dst)
    h1 = mlp0(agg0.reshape(2, NPAD, LANE), x_t.reshape(2, N, LANE),
              W1_0, b1_0.reshape(1, H), W2_0, b2_0.reshape(1, H))
    agg1 = seg4(h1.reshape(4 * N, LANE), sidx4, dst)
    h2 = mlp1(agg1.reshape(4, NPAD, LANE), h1,
              W1_1, b1_1.reshape(1, H), W2_1, b2_1.reshape(1, H))
    agg2 = seg4(h2.reshape(4 * N, LANE), sidx4, dst)
    out = mlp2(agg2.reshape(4, NPAD, LANE), h2,
               W1_2, b1_2.reshape(1, H), W2_2, b2_2.reshape(1, OUT))
    return out


# speedup vs baseline: 1.0501x; 1.0016x over previous
import functools

import jax
import jax.numpy as jnp
from jax import lax
from jax.experimental import pallas as pl
from jax.experimental.pallas import tpu as pltpu
from jax.experimental.pallas import tpu_sc as plsc

N = 10000
E = 160000
D = 256
H = 512
OUT = 256

NC = 2
NS = 16
LANE = 128

EDGES_PER_TILE = E // NS
K = 125
NCH = EDGES_PER_TILE // K
NGRP = 2
M = NCH // NGRP
NPAD = 10240
ROWS_PER_TILE = NPAD // NS
ZR = 32


def _sc_segment_sum(cb):
    cbh = cb // NC

    mesh = plsc.VectorSubcoreMesh(core_axis_name="c", subcore_axis_name="s",
                                  num_cores=NC, num_subcores=NS)

    @functools.partial(
        pl.kernel,
        out_type=jax.ShapeDtypeStruct((cb * NPAD, LANE), jnp.float32),
        mesh=mesh,
        scratch_types=[
            pltpu.VMEM((M, K), jnp.int32),
            pltpu.VMEM((M, K), jnp.int32),
            pltpu.VMEM((2, K, LANE), jnp.float32),
            pltpu.VMEM((ZR, LANE), jnp.float32),
            pltpu.VMEM_SHARED((NPAD, LANE), jnp.float32),
            pltpu.SemaphoreType.DMA,
            pltpu.SemaphoreType.DMA,
        ],
    )
    def seg_kernel(ftab, sidx, didx, agg_out, src_v, dst_v, rows_v, zero_v,
                   agg_s, gsA, gsB):
        c = lax.axis_index("c")
        s = lax.axis_index("s")

        def zrow(i, _):
            def zcol(k, _):
                zero_v[i, pl.ds(k * 16, 16)] = jnp.zeros((16,), jnp.float32)
                return 0
            return lax.fori_loop(0, LANE // 16, zcol, 0)
        lax.fori_loop(0, ZR, zrow, 0)

        for b_i in range(cbh):
            idx_row = (c * cbh + b_i) * NS + s

            for z in range(ROWS_PER_TILE // ZR):
                pltpu.async_copy(
                    zero_v, agg_s.at[pl.ds(s * ROWS_PER_TILE + z * ZR, ZR)],
                    gsA)
            for z in range(ROWS_PER_TILE // ZR):
                pltpu.make_async_copy(
                    zero_v, agg_s.at[pl.ds(s * ROWS_PER_TILE, ZR)],
                    gsA).wait()
            plsc.subcore_barrier()

            for g in range(NGRP):
                pltpu.sync_copy(sidx.at[idx_row * NGRP + g], src_v)
                pltpu.sync_copy(didx.at[s * NGRP + g], dst_v)

                bufA = rows_v.at[0]
                bufB = rows_v.at[1]
                pltpu.async_copy(ftab.at[src_v.at[0]], bufA, gsA)

                def pair(i, _):
                    j = 2 * i
                    pltpu.make_async_copy(
                        ftab.at[src_v.at[0]], bufA, gsA).wait()
                    pltpu.async_copy(ftab.at[src_v.at[j + 1]], bufB, gsB)
                    pltpu.sync_copy(bufA, agg_s.at[dst_v.at[j]], add=True)
                    pltpu.make_async_copy(
                        ftab.at[src_v.at[0]], bufB, gsB).wait()
                    jn = jnp.minimum(j + 2, M - 1)
                    pltpu.async_copy(ftab.at[src_v.at[jn]], bufA, gsA)
                    pltpu.sync_copy(bufB, agg_s.at[dst_v.at[j + 1]], add=True)
                    return 0
                lax.fori_loop(0, M // 2, pair, 0)
                pltpu.make_async_copy(ftab.at[src_v.at[0]], bufA, gsA).wait()
            plsc.subcore_barrier()

            blk = c + NC * b_i
            pltpu.sync_copy(
                agg_s.at[pl.ds(s * ROWS_PER_TILE, ROWS_PER_TILE)],
                agg_out.at[pl.ds(blk * NPAD + s * ROWS_PER_TILE,
                                 ROWS_PER_TILE)])
            if b_i + 1 < cbh:
                plsc.subcore_barrier()

    return seg_kernel


def _tc_mlp(cb_in, d_in, h_out, cb_out, rows, h_blocked=True):
    def body(agg_ref, h_ref, w1_ref, b1_ref, w2_ref, b2_ref, out_ref):
        bf = jnp.bfloat16
        acc = jnp.broadcast_to(b1_ref[0], (rows, H))
        for b in range(cb_in):
            acc = acc + jnp.dot(agg_ref[b].astype(bf),
                                w1_ref[b * LANE:(b + 1) * LANE, :].astype(bf),
                                preferred_element_type=jnp.float32)
        for b in range(cb_in):
            hb = h_ref[b] if h_blocked else h_ref[:, b * LANE:(b + 1) * LANE]
            acc = acc + jnp.dot(
                hb.astype(bf),
                w1_ref[d_in + b * LANE:d_in + (b + 1) * LANE, :].astype(bf),
                preferred_element_type=jnp.float32)
        y = jnp.maximum(acc, 0.0)
        out = jnp.dot(y.astype(bf), w2_ref[...].astype(bf),
                      preferred_element_type=jnp.float32) + b2_ref[0]
        if cb_out:
            for b in range(cb_out):
                out_ref[b] = out[:, b * LANE:(b + 1) * LANE]
        else:
            out_ref[...] = out

    if cb_out:
        out_shape = jax.ShapeDtypeStruct((cb_out, N, LANE), jnp.float32)
        out_spec = pl.BlockSpec((cb_out, rows, LANE), lambda i: (0, i, 0))
    else:
        out_shape = jax.ShapeDtypeStruct((N, h_out), jnp.float32)
        out_spec = pl.BlockSpec((rows, h_out), lambda i: (i, 0))

    return pl.pallas_call(
        body,
        grid=(N // rows,),
        in_specs=[
            pl.BlockSpec((cb_in, rows, LANE), lambda i: (0, i, 0)),
            (pl.BlockSpec((cb_in, rows, LANE), lambda i: (0, i, 0))
             if h_blocked else pl.BlockSpec((rows, d_in), lambda i: (i, 0))),
            pl.BlockSpec((2 * d_in, H), lambda i: (0, 0)),
            pl.BlockSpec((1, H), lambda i: (0, 0)),
            pl.BlockSpec((H, h_out), lambda i: (0, 0)),
            pl.BlockSpec((1, h_out), lambda i: (0, 0)),
        ],
        out_specs=out_spec,
        out_shape=out_shape,
    )


def kernel(x, edge_index, W1_0, b1_0, W2_0, b2_0, W1_1, b1_1, W2_1, b2_1,
           W1_2, b1_2, W2_2, b2_2):
    src = edge_index[0].reshape(NS, NCH, K)
    dst = edge_index[1].reshape(NS, NCH, K)

    sidx2 = jnp.concatenate([src * 2, src * 2 + 1], axis=0).reshape(-1, M, K)
    sidx4 = jnp.concatenate([src, src + 2 * N, src + N, src + 3 * N],
                            axis=0).reshape(-1, M, K)
    dst = dst.reshape(-1, M, K)

    seg2 = _sc_segment_sum(2)
    seg4 = _sc_segment_sum(4)
    mlp0 = _tc_mlp(2, D, H, 4, 2000, h_blocked=False)
    mlp1 = _tc_mlp(4, H, H, 4, 2000)
    mlp2 = _tc_mlp(4, H, OUT, 0, 2000)

    agg0 = seg2(x.reshape(2 * N, LANE), sidx2, dst)
    h1 = mlp0(agg0.reshape(2, NPAD, LANE), x,
              W1_0, b1_0.reshape(1, H), W2_0, b2_0.reshape(1, H))
    agg1 = seg4(h1.reshape(4 * N, LANE), sidx4, dst)
    h2 = mlp1(agg1.reshape(4, NPAD, LANE), h1,
              W1_1, b1_1.reshape(1, H), W2_1, b2_1.reshape(1, H))
    agg2 = seg4(h2.reshape(4 * N, LANE), sidx4, dst)
    out = mlp2(agg2.reshape(4, NPAD, LANE), h2,
               W1_2, b1_2.reshape(1, H), W2_2, b2_2.reshape(1, OUT))
    return out
